# baseline (speedup 1.0000x reference)
import functools

import jax
import jax.numpy as jnp
from jax.experimental import pallas as pl
from jax.experimental.pallas import tpu as pltpu

_LANE = 128


def _codec_body(groups, btiles, sp_ref, e_ref, d_ref, eb_ref, out_ref):
    tm = out_ref.shape[0]
    ones = jnp.ones((tm, 64), dtype=jnp.bfloat16)
    for tb in btiles:
        out_ref[:, tb:tb + _LANE] = jnp.zeros((tm, _LANE), jnp.float32)
    for i, (a0, a1, ub0, ub1, ho) in enumerate(groups):
        w = a1 - a0
        sp_r = sp_ref[:, a0:a1].astype(jnp.bfloat16)
        e_r = e_ref[:, ho:ho + w]
        tok = jax.lax.dot_general(
            sp_r, e_r, (((1,), (1,)), ((), ())),
            preferred_element_type=jnp.float32)
        tok = (tok + eb_ref[i:i + 1, :]).astype(jnp.bfloat16)
        tok = jnp.concatenate([tok, ones], axis=1)
        d_r = d_ref[ho:ho + w, :]
        rec = jax.lax.dot_general(
            tok, d_r, (((1,), (1,)), ((), ())),
            preferred_element_type=jnp.float32)
        if ub0 > ub1:
            out_ref[:, a0:a1] += rec
            continue
        if ub1 > ub0:
            out_ref[:, ub0:ub1] = rec[:, ub0 - a0:ub1 - a0]
        if ub0 > a0:
            out_ref[:, a0:ub0] += rec[:, :ub0 - a0]
        if a1 > ub1:
            out_ref[:, ub1:a1] += rec[:, ub1 - a0:]


def kernel(spikes, neuron_regions, eids, enc_w, enc_b, dec_w, dec_b):
    B, T, N = spikes.shape
    M = B * T
    D = enc_w[0].shape[0]
    R = len(enc_w)
    sizes = [wt.shape[1] for wt in enc_w]
    offs = [0]
    for n in sizes:
        offs.append(offs[-1] + n)

    eh, dh, groups, btiles = [], [], [], []
    ho = 0
    for i in range(R):
        off, n = offs[i], sizes[i]
        a0 = (off // _LANE) * _LANE
        a1 = -(-(off + n) // _LANE) * _LANE
        ub0 = -(-off // _LANE) * _LANE
        ub1 = ((off + n) // _LANE) * _LANE
        lo, hi = off - a0, a1 - (off + n)
        eh.append(jnp.pad(enc_w[i], ((0, 0), (lo, hi))).astype(jnp.bfloat16))
        dwp = jnp.pad(dec_w[i], ((lo, hi), (0, 0)))
        dbp = jnp.pad(dec_b[i][:, None], ((lo, hi), (0, 0)))
        dh.append(jnp.pad(jnp.concatenate([dwp, dbp], axis=1),
                          ((0, 0), (0, 63))).astype(jnp.bfloat16))
        groups.append((a0, a1, ub0, ub1, ho))
        if off % _LANE:
            btiles.append(a0)
        ho += a1 - a0

    sp2 = spikes.reshape(M, N)
    E = jnp.concatenate(eh, axis=1)
    Dc = jnp.concatenate(dh, axis=0)
    EB = jnp.stack(enc_b, axis=0)
    W = E.shape[1]

    TM = 160
    grid = (M // TM,)
    out = pl.pallas_call(
        functools.partial(_codec_body, tuple(groups), tuple(btiles)),
        grid=grid,
        in_specs=[
            pl.BlockSpec((TM, N), lambda i: (i, 0)),
            pl.BlockSpec((D, W), lambda i: (0, 0)),
            pl.BlockSpec((W, 128), lambda i: (0, 0)),
            pl.BlockSpec(EB.shape, lambda i: (0, 0)),
        ],
        out_specs=pl.BlockSpec((TM, N), lambda i: (i, 0)),
        out_shape=jax.ShapeDtypeStruct((M, N), spikes.dtype),
        compiler_params=pltpu.CompilerParams(
            dimension_semantics=("arbitrary",)),
    )(sp2, E, Dc, EB)
    return out.reshape(B, T, N)


# speedup vs baseline: 1.3135x; 1.3135x over previous
import functools

import jax
import jax.numpy as jnp
from jax.experimental import pallas as pl
from jax.experimental.pallas import tpu as pltpu

_LANE = 128


def _codec_body(groups, btiles, sp_ref, e_ref, d_ref, eb_ref, db_ref, out_ref):
    tm = out_ref.shape[0]
    for tb in btiles:
        out_ref[:, tb:tb + _LANE] = jnp.broadcast_to(
            db_ref[:, tb:tb + _LANE], (tm, _LANE))
    toks = []
    for i, (a0, a1, ub0, ub1, ho) in enumerate(groups):
        w = a1 - a0
        sp_r = sp_ref[:, a0:a1].astype(jnp.bfloat16)
        e_r = e_ref[:, ho:ho + w]
        tok = jax.lax.dot_general(
            sp_r, e_r, (((1,), (1,)), ((), ())),
            preferred_element_type=jnp.float32)
        toks.append((tok + eb_ref[i:i + 1, :]).astype(jnp.bfloat16))
    for i, (a0, a1, ub0, ub1, ho) in enumerate(groups):
        w = a1 - a0
        d_r = d_ref[ho:ho + w, :]
        rec = jax.lax.dot_general(
            toks[i], d_r, (((1,), (1,)), ((), ())),
            preferred_element_type=jnp.float32)
        if ub0 > ub1:
            out_ref[:, a0:a1] += rec
            continue
        if ub1 > ub0:
            out_ref[:, ub0:ub1] = (rec[:, ub0 - a0:ub1 - a0]
                                   + db_ref[:, ub0:ub1])
        if ub0 > a0:
            out_ref[:, a0:ub0] += rec[:, :ub0 - a0]
        if a1 > ub1:
            out_ref[:, ub1:a1] += rec[:, ub1 - a0:]


def kernel(spikes, neuron_regions, eids, enc_w, enc_b, dec_w, dec_b):
    B, T, N = spikes.shape
    M = B * T
    D = enc_w[0].shape[0]
    R = len(enc_w)
    sizes = [wt.shape[1] for wt in enc_w]
    offs = [0]
    for n in sizes:
        offs.append(offs[-1] + n)

    eh, dh, groups, btiles = [], [], [], []
    ho = 0
    for i in range(R):
        off, n = offs[i], sizes[i]
        a0 = (off // _LANE) * _LANE
        a1 = -(-(off + n) // _LANE) * _LANE
        ub0 = -(-off // _LANE) * _LANE
        ub1 = ((off + n) // _LANE) * _LANE
        lo, hi = off - a0, a1 - (off + n)
        eh.append(jnp.pad(enc_w[i], ((0, 0), (lo, hi))).astype(jnp.bfloat16))
        dh.append(jnp.pad(dec_w[i], ((lo, hi), (0, 0))).astype(jnp.bfloat16))
        groups.append((a0, a1, ub0, ub1, ho))
        if off % _LANE:
            btiles.append(a0)
        ho += a1 - a0

    sp2 = spikes.reshape(M, N)
    E = jnp.concatenate(eh, axis=1)
    Dc = jnp.concatenate(dh, axis=0)
    EB = jnp.stack(enc_b, axis=0)
    DB = jnp.concatenate(dec_b)[None, :]
    W = E.shape[1]

    TM = 160
    grid = (M // TM,)
    out = pl.pallas_call(
        functools.partial(_codec_body, tuple(groups), tuple(btiles)),
        grid=grid,
        in_specs=[
            pl.BlockSpec((TM, N), lambda i: (i, 0)),
            pl.BlockSpec((D, W), lambda i: (0, 0)),
            pl.BlockSpec((W, D), lambda i: (0, 0)),
            pl.BlockSpec(EB.shape, lambda i: (0, 0)),
            pl.BlockSpec((1, N), lambda i: (0, 0)),
        ],
        out_specs=pl.BlockSpec((TM, N), lambda i: (i, 0)),
        out_shape=jax.ShapeDtypeStruct((M, N), spikes.dtype),
        compiler_params=pltpu.CompilerParams(
            dimension_semantics=("arbitrary",)),
    )(sp2, E, Dc, EB, DB)
    return out.reshape(B, T, N)


# speedup vs baseline: 1.3181x; 1.0035x over previous
import functools

import jax
import jax.numpy as jnp
from jax.experimental import pallas as pl
from jax.experimental.pallas import tpu as pltpu

_LANE = 128


def _codec_body(groups, btiles, sp_ref, e_ref, d_ref, eb_ref, db_ref, out_ref):
    tm = out_ref.shape[0]
    for tb in btiles:
        out_ref[:, tb:tb + _LANE] = jnp.broadcast_to(
            db_ref[:, tb:tb + _LANE], (tm, _LANE))
    toks = []
    for i, (a0, a1, ub0, ub1, ho) in enumerate(groups):
        w = a1 - a0
        sp_r = sp_ref[:, a0:a1].astype(jnp.bfloat16)
        e_r = e_ref[:, ho:ho + w]
        tok = jax.lax.dot_general(
            sp_r, e_r, (((1,), (1,)), ((), ())),
            preferred_element_type=jnp.float32)
        toks.append((tok + eb_ref[i:i + 1, :]).astype(jnp.bfloat16))
    for i, (a0, a1, ub0, ub1, ho) in enumerate(groups):
        w = a1 - a0
        d_r = d_ref[ho:ho + w, :]
        rec = jax.lax.dot_general(
            toks[i], d_r, (((1,), (1,)), ((), ())),
            preferred_element_type=jnp.float32)
        if ub0 > ub1:
            out_ref[:, a0:a1] += rec
            continue
        if ub1 > ub0:
            out_ref[:, ub0:ub1] = (rec[:, ub0 - a0:ub1 - a0]
                                   + db_ref[:, ub0:ub1])
        if ub0 > a0:
            out_ref[:, a0:ub0] += rec[:, :ub0 - a0]
        if a1 > ub1:
            out_ref[:, ub1:a1] += rec[:, ub1 - a0:]


def kernel(spikes, neuron_regions, eids, enc_w, enc_b, dec_w, dec_b):
    B, T, N = spikes.shape
    M = B * T
    D = enc_w[0].shape[0]
    R = len(enc_w)
    sizes = [wt.shape[1] for wt in enc_w]
    offs = [0]
    for n in sizes:
        offs.append(offs[-1] + n)

    eh, dh, groups, btiles = [], [], [], []
    ho = 0
    for i in range(R):
        off, n = offs[i], sizes[i]
        a0 = (off // _LANE) * _LANE
        a1 = -(-(off + n) // _LANE) * _LANE
        ub0 = -(-off // _LANE) * _LANE
        ub1 = ((off + n) // _LANE) * _LANE
        lo, hi = off - a0, a1 - (off + n)
        eh.append(jnp.pad(enc_w[i], ((0, 0), (lo, hi))).astype(jnp.bfloat16))
        dh.append(jnp.pad(dec_w[i], ((lo, hi), (0, 0))).astype(jnp.bfloat16))
        groups.append((a0, a1, ub0, ub1, ho))
        if off % _LANE:
            btiles.append(a0)
        ho += a1 - a0

    sp2 = spikes.reshape(M, N)
    E = jnp.concatenate(eh, axis=1)
    Dc = jnp.concatenate(dh, axis=0)
    EB = jnp.stack(enc_b, axis=0)
    DB = jnp.concatenate(dec_b)[None, :]
    W = E.shape[1]

    TM = 128
    grid = (pl.cdiv(M, TM),)
    out = pl.pallas_call(
        functools.partial(_codec_body, tuple(groups), tuple(btiles)),
        grid=grid,
        in_specs=[
            pl.BlockSpec((TM, N), lambda i: (i, 0)),
            pl.BlockSpec((D, W), lambda i: (0, 0)),
            pl.BlockSpec((W, D), lambda i: (0, 0)),
            pl.BlockSpec(EB.shape, lambda i: (0, 0)),
            pl.BlockSpec((1, N), lambda i: (0, 0)),
        ],
        out_specs=pl.BlockSpec((TM, N), lambda i: (i, 0)),
        out_shape=jax.ShapeDtypeStruct((M, N), spikes.dtype),
        compiler_params=pltpu.CompilerParams(
            dimension_semantics=("arbitrary",)),
    )(sp2, E, Dc, EB, DB)
    return out.reshape(B, T, N)
